# baseline (speedup 1.0000x reference)
import functools

import jax
import jax.numpy as jnp
from jax import lax
from jax.experimental import pallas as pl
from jax.experimental.pallas import tpu as pltpu
from jax.experimental.pallas import tpu_sc as plsc


def _sc_add(B, S, D):
    NC, NS = 2, 16
    NW = NC * NS
    SW = S // NW
    C = 32
    n_chunks = SW // C
    n_steps = n_chunks * B
    NBUF = 2

    mesh = plsc.VectorSubcoreMesh(core_axis_name="c", subcore_axis_name="s")

    @functools.partial(
        pl.kernel,
        mesh=mesh,
        out_type=jax.ShapeDtypeStruct((B * S, D), jnp.float32),
        scratch_types=[
            pltpu.VMEM((C, D), jnp.float32),
            pltpu.VMEM((NBUF, C, D), jnp.float32),
            pltpu.SemaphoreType.DMA,
            pltpu.SemaphoreType.DMA,
            pltpu.SemaphoreType.DMA,
        ],
    )
    def run(x_hbm, pos_hbm, out_hbm, p_v, x_v, sem_in, sem_pos, sem_out):
        wid = lax.axis_index("s") * NC + lax.axis_index("c")
        s_base = wid * SW

        def row0(t):
            c, b = t // B, t % B
            return b * S + s_base + c * C

        def start_in(t):
            pltpu.async_copy(x_hbm.at[pl.ds(row0(t), C)], x_v.at[t % NBUF], sem_in)

        def start_pos(c):
            pltpu.async_copy(pos_hbm.at[pl.ds(s_base + c * C, C)], p_v, sem_pos)

        def wait(src, dst, sem):
            pltpu.make_async_copy(src, dst, sem).wait()

        start_pos(0)
        start_in(0)
        outs_waited = 0
        for t in range(n_steps):
            c, b = t // B, t % B
            if b == 0:
                wait(pos_hbm.at[pl.ds(0, C)], p_v, sem_pos)
            wait(x_hbm.at[pl.ds(0, C)], x_v.at[t % NBUF], sem_in)
            if t + 1 < n_steps:
                if t >= 1:
                    wait(x_v.at[0], out_hbm.at[pl.ds(0, C)], sem_out)
                    outs_waited += 1
                start_in(t + 1)

            xb = x_v.at[t % NBUF]

            def add_body(i, acc):
                r = i // 8
                j = (i % 8) * 128
                vals = [p_v[r, pl.ds(j + k * 16, 16)] for k in range(8)]
                for k in range(8):
                    plsc.addupdate(xb.at[r, pl.ds(j + k * 16, 16)], vals[k])
                return acc

            lax.fori_loop(0, C * 8, add_body, 0)

            if b == B - 1 and c + 1 < n_chunks:
                start_pos(c + 1)
            pltpu.async_copy(xb, out_hbm.at[pl.ds(row0(t), C)], sem_out)
        for _ in range(n_steps - outs_waited):
            wait(x_v.at[0], out_hbm.at[pl.ds(0, C)], sem_out)

    return run


def kernel(x, pos_weight):
    B, S, D = x.shape
    out = _sc_add(B, S, D)(x.reshape(B * S, D), pos_weight[:S])
    return out.reshape(B, S, D)


# speedup vs baseline: 1.2111x; 1.2111x over previous
import functools

import jax
import jax.numpy as jnp
from jax import lax
from jax.experimental import pallas as pl
from jax.experimental.pallas import tpu as pltpu
from jax.experimental.pallas import tpu_sc as plsc


def _sc_add(B, S, D):
    NC, NS = 2, 16
    NW = NC * NS
    SW = S // NW
    C = 16
    n_chunks = SW // C
    n_steps = n_chunks * B
    NBUF = 5
    LOOKAHEAD = 3

    mesh = plsc.VectorSubcoreMesh(core_axis_name="c", subcore_axis_name="s")

    @functools.partial(
        pl.kernel,
        mesh=mesh,
        out_type=jax.ShapeDtypeStruct((B * S, D), jnp.float32),
        scratch_types=[
            pltpu.VMEM((2, C, D), jnp.float32),
            pltpu.VMEM((NBUF, C, D), jnp.float32),
            pltpu.SemaphoreType.DMA,
            pltpu.SemaphoreType.DMA,
            pltpu.SemaphoreType.DMA,
        ],
    )
    def run(x_hbm, pos_hbm, out_hbm, p_v, x_v, sem_in, sem_pos, sem_out):
        wid = lax.axis_index("s") * NC + lax.axis_index("c")
        s_base = wid * SW

        def row0(t):
            c, b = t // B, t % B
            return b * S + s_base + c * C

        def start_in(t):
            pltpu.async_copy(x_hbm.at[pl.ds(row0(t), C)], x_v.at[t % NBUF], sem_in)

        def start_pos(c):
            pltpu.async_copy(
                pos_hbm.at[pl.ds(s_base + c * C, C)], p_v.at[c % 2], sem_pos
            )

        def wait(src, dst, sem):
            pltpu.make_async_copy(src, dst, sem).wait()

        start_pos(0)
        for t in range(LOOKAHEAD):
            start_in(t)
        outs_waited = 0
        for t in range(n_steps):
            c = t // B
            if t % B == 0 and c + 1 < n_chunks:
                start_pos(c + 1)
            if t % B == 0:
                wait(pos_hbm.at[pl.ds(0, C)], p_v.at[c % 2], sem_pos)
            wait(x_hbm.at[pl.ds(0, C)], x_v.at[t % NBUF], sem_in)
            if t + LOOKAHEAD < n_steps:
                if t + LOOKAHEAD - NBUF >= 0:
                    wait(x_v.at[0], out_hbm.at[pl.ds(0, C)], sem_out)
                    outs_waited += 1
                start_in(t + LOOKAHEAD)

            xb = x_v.at[t % NBUF]
            pb = p_v.at[c % 2]

            def add_body(i, acc):
                r = i // 8
                j = (i % 8) * 128
                vals = [pb[r, pl.ds(j + k * 16, 16)] for k in range(8)]
                for k in range(8):
                    plsc.addupdate(xb.at[r, pl.ds(j + k * 16, 16)], vals[k])
                return acc

            lax.fori_loop(0, C * 8, add_body, 0)

            pltpu.async_copy(xb, out_hbm.at[pl.ds(row0(t), C)], sem_out)
        for _ in range(n_steps - outs_waited):
            wait(x_v.at[0], out_hbm.at[pl.ds(0, C)], sem_out)

    return run


def kernel(x, pos_weight):
    B, S, D = x.shape
    out = _sc_add(B, S, D)(x.reshape(B * S, D), pos_weight[:S])
    return out.reshape(B, S, D)
